# baseline (speedup 1.0000x reference)
import functools

import jax
import jax.numpy as jnp
from jax import lax
from jax.experimental import pallas as pl
from jax.experimental.pallas import tpu as pltpu
from jax.experimental.pallas import tpu_sc as plsc

NUM_TABLES = 26
VOCAB_SZ = 100000
BATCH_SZ = 16384

NUM_CORES = 2
NUM_SUBCORES = 16
LANES = 16
NUM_WORKERS = NUM_CORES * NUM_SUBCORES

TOTAL = BATCH_SZ * NUM_TABLES
BLOCK = TOTAL // NUM_WORKERS
NVEC = BLOCK // LANES
PERIOD = 13


def _emb_body(tables_hbm, idx_hbm, out_hbm, idx_v, out_v, off_v, sem):
    wid = lax.axis_index("s") * NUM_CORES + lax.axis_index("c")
    base = wid * BLOCK

    pltpu.sync_copy(idx_hbm.at[pl.ds(base, BLOCK)], idx_v)

    lane = lax.iota(jnp.int32, LANES)
    for j in range(PERIOD):
        pos = lane + j * LANES
        off_v[pl.ds(j * LANES, LANES)] = lax.rem(pos, NUM_TABLES) * VOCAB_SZ

    @pl.loop(0, NVEC, unroll=8)
    def _flatten(i):
        sl = pl.ds(i * LANES, LANES)
        j = lax.rem(i, PERIOD)
        idx_v[sl] = idx_v[sl] + off_v[pl.ds(j * LANES, LANES)]

    pltpu.async_copy(tables_hbm.at[idx_v], out_v, sem).wait()

    pltpu.sync_copy(out_v, out_hbm.at[pl.ds(base, BLOCK)])


@functools.partial(
    pl.kernel,
    out_type=jax.ShapeDtypeStruct((TOTAL,), jnp.float32),
    mesh=plsc.VectorSubcoreMesh(core_axis_name="c", subcore_axis_name="s"),
    scratch_types=[
        pltpu.VMEM((BLOCK,), jnp.int32),
        pltpu.VMEM((BLOCK,), jnp.float32),
        pltpu.VMEM((PERIOD * LANES,), jnp.int32),
        pltpu.SemaphoreType.DMA,
    ],
    compiler_params=pltpu.CompilerParams(needs_layout_passes=False),
)
def _emb_kernel(tables_hbm, idx_hbm, out_hbm, idx_v, out_v, off_v, sem):
    _emb_body(tables_hbm, idx_hbm, out_hbm, idx_v, out_v, off_v, sem)


def kernel(categorical_tensor, tables):
    idx_flat = categorical_tensor.astype(jnp.int32).reshape(TOTAL)
    tables_flat = tables.reshape(NUM_TABLES * VOCAB_SZ)
    out_flat = _emb_kernel(tables_flat, idx_flat)
    return out_flat.reshape(BATCH_SZ, NUM_TABLES)
